# baseline (speedup 1.0000x reference)
import functools

import jax
import jax.numpy as jnp
from jax import lax
from jax.experimental import pallas as pl
from jax.experimental.pallas import tpu as pltpu
from jax.experimental.pallas import tpu_sc as plsc

VOCAB = 1_000_000
EMBED = 64
CTX = 200
HALF = VOCAB // 2
IDX_PAD = 224
S_BLK = 1024
N_BLK = 65_536


def _sc_multihot_body(idx_hbm, ones_hbm, zeros_hbm, out_hbm,
                      idx_v, idxe_v, ones_v, spm, _sem):
    c = lax.axis_index("c")
    sid = lax.axis_index("s")

    @pl.when(sid == 0)
    def _():
        pltpu.sync_copy(zeros_hbm, spm.at[pl.ds(0, HALF)])
        pltpu.sync_copy(idx_hbm, idx_v)
        pltpu.sync_copy(ones_hbm, ones_v)
        base = c * HALF
        for j in range(2):
            for k in range(IDX_PAD // 32):
                sl = pl.ds(16 * k, 16)
                v = idx_v[j, sl]
                local = v - base
                valid = jnp.logical_and(local >= 0, local < HALF)
                idxe_v[j, sl] = jnp.where(valid, local, HALF)
        for j in range(2):
            pltpu.sync_copy(ones_v.at[j], spm.at[idxe_v.at[j]], add=True)
        pltpu.sync_copy(spm.at[pl.ds(0, HALF)], out_hbm.at[pl.ds(base, HALF)])


def _pool_body(bids_sm, valid_sm, s_ref, et_ref, out_ref):
    i = pl.program_id(0)
    bid = bids_sm[i]
    limit = jnp.minimum(VOCAB - bid * S_BLK, S_BLK)
    lane = lax.broadcasted_iota(jnp.int32, (1, S_BLK), 1)
    sv = s_ref[...].reshape(1, S_BLK)
    sv = jnp.where(lane < limit, sv, 0.0) * valid_sm[i].astype(jnp.float32)
    lane2 = lax.broadcasted_iota(jnp.int32, (EMBED, S_BLK), 1)
    ev = jnp.where(lane2 < limit, et_ref[...], 0.0)
    z = lax.dot_general(sv, ev, (((1,), (1,)), ((), ())),
                        preferred_element_type=jnp.float32)

    @pl.when(i == 0)
    def _():
        out_ref[...] = z

    @pl.when(i > 0)
    def _():
        out_ref[...] += z


def _project_body(sum_ref, wt_ref, b_ref, out_ref):
    avg = sum_ref[...] * (1.0 / CTX)
    out_ref[...] = lax.dot_general(
        avg, wt_ref[...], (((1,), (0,)), ((), ())),
        preferred_element_type=jnp.float32) + b_ref[...].reshape(1, N_BLK)


def kernel(context_idx, embeddings, W, b):
    ci = context_idx.astype(jnp.int32)
    idx2 = jnp.pad(ci, (0, IDX_PAD - CTX),
                   constant_values=VOCAB).reshape(2, IDX_PAD // 2)
    ones2 = jnp.ones((2, IDX_PAD // 2), jnp.float32)
    zeros_half = jnp.zeros((HALF,), jnp.float32)

    mesh = plsc.VectorSubcoreMesh(core_axis_name="c", subcore_axis_name="s")
    multihot = pl.kernel(
        _sc_multihot_body,
        mesh=mesh,
        out_type=jax.ShapeDtypeStruct((VOCAB,), jnp.float32),
        scratch_types=[
            pltpu.VMEM((2, IDX_PAD // 2), jnp.int32),
            pltpu.VMEM((2, IDX_PAD // 2), jnp.int32),
            pltpu.VMEM((2, IDX_PAD // 2), jnp.float32),
            pltpu.VMEM_SHARED((HALF + 8,), jnp.float32),
            pltpu.SemaphoreType.DMA,
        ],
        compiler_params=pltpu.CompilerParams(use_tc_tiling_on_sc=False),
    )
    s = multihot(idx2, ones2, zeros_half)

    eT = embeddings.T
    wT = W.T

    bids, counts = jnp.unique(ci // S_BLK, size=CTX, fill_value=0,
                              return_counts=True)
    bids = bids.astype(jnp.int32)
    valid = (counts > 0).astype(jnp.int32)

    grid_spec = pltpu.PrefetchScalarGridSpec(
        num_scalar_prefetch=2,
        grid=(CTX,),
        in_specs=[
            pl.BlockSpec((S_BLK,), lambda i, bids, valid: (bids[i],)),
            pl.BlockSpec((EMBED, S_BLK), lambda i, bids, valid: (0, bids[i])),
        ],
        out_specs=pl.BlockSpec((1, EMBED), lambda i, bids, valid: (0, 0)),
    )
    sum_embed = pl.pallas_call(
        _pool_body,
        grid_spec=grid_spec,
        out_shape=jax.ShapeDtypeStruct((1, EMBED), jnp.float32),
    )(bids, valid, s, eT)

    logits = pl.pallas_call(
        _project_body,
        grid=(pl.cdiv(VOCAB, N_BLK),),
        in_specs=[
            pl.BlockSpec((1, EMBED), lambda i: (0, 0)),
            pl.BlockSpec((EMBED, N_BLK), lambda i: (0, i)),
            pl.BlockSpec((N_BLK,), lambda i: (i,)),
        ],
        out_specs=pl.BlockSpec((1, N_BLK), lambda i: (0, i)),
        out_shape=jax.ShapeDtypeStruct((1, VOCAB), jnp.float32),
    )(sum_embed, wT, b)
    return logits


# speedup vs baseline: 1.2065x; 1.2065x over previous
import functools

import jax
import jax.numpy as jnp
from jax import lax
from jax.experimental import pallas as pl
from jax.experimental.pallas import tpu as pltpu
from jax.experimental.pallas import tpu_sc as plsc

VOCAB = 1_000_000
EMBED = 64
CTX = 200
HALF = VOCAB // 2
IDX_PAD = 224
BLK = 32_768
NBK = 31


def _sc_multihot_body(idx_hbm, ones_hbm, zeros_hbm, out_hbm,
                      idx_v, idxe_v, ones_v, spm, _sem):
    c = lax.axis_index("c")
    sid = lax.axis_index("s")

    @pl.when(sid == 0)
    def _():
        pltpu.sync_copy(zeros_hbm, spm.at[pl.ds(0, HALF)])
        pltpu.sync_copy(idx_hbm, idx_v)
        pltpu.sync_copy(ones_hbm, ones_v)
        base = c * HALF
        for j in range(2):
            for k in range(IDX_PAD // 32):
                sl = pl.ds(16 * k, 16)
                v = idx_v[j, sl]
                local = v - base
                valid = jnp.logical_and(local >= 0, local < HALF)
                idxe_v[j, sl] = jnp.where(valid, local, HALF)
        for j in range(2):
            pltpu.sync_copy(ones_v.at[j], spm.at[idxe_v.at[j]], add=True)
        pltpu.sync_copy(spm.at[pl.ds(0, HALF)], out_hbm.at[pl.ds(base, HALF)])


def _pool_project_body(s_ref, et_ref, wt_ref, b_ref, out_ref, acc_ref):
    i = pl.program_id(0)

    @pl.when(i < NBK)
    def _():
        sv = s_ref[...].reshape(1, BLK)
        limit = jnp.minimum(VOCAB - i * BLK, BLK)
        lane = lax.broadcasted_iota(jnp.int32, (1, BLK), 1)
        sv = jnp.where(lane < limit, sv, 0.0)

        def dot(ev):
            return lax.dot_general(sv, ev, (((1,), (1,)), ((), ())),
                                   preferred_element_type=jnp.float32)

        @pl.when(i == 0)
        def _():
            acc_ref[...] = dot(et_ref[...])

        @pl.when(jnp.logical_and(i > 0, i < NBK - 1))
        def _():
            acc_ref[...] += dot(et_ref[...])

        @pl.when(i == NBK - 1)
        def _():
            lane2 = lax.broadcasted_iota(jnp.int32, (EMBED, BLK), 1)
            ev = jnp.where(lane2 < limit, et_ref[...], 0.0)
            acc_ref[...] += dot(ev)

    @pl.when(i >= NBK)
    def _():
        avg = acc_ref[...] * (1.0 / CTX)
        out_ref[...] = lax.dot_general(
            avg, wt_ref[...], (((1,), (0,)), ((), ())),
            preferred_element_type=jnp.float32) + b_ref[...].reshape(1, BLK)


def kernel(context_idx, embeddings, W, b):
    ci = context_idx.astype(jnp.int32)
    idx2 = jnp.pad(ci, (0, IDX_PAD - CTX),
                   constant_values=VOCAB).reshape(2, IDX_PAD // 2)
    ones2 = jnp.ones((2, IDX_PAD // 2), jnp.float32)
    zeros_half = jnp.zeros((HALF,), jnp.float32)

    mesh = plsc.VectorSubcoreMesh(core_axis_name="c", subcore_axis_name="s")
    multihot = pl.kernel(
        _sc_multihot_body,
        mesh=mesh,
        out_type=jax.ShapeDtypeStruct((VOCAB,), jnp.float32),
        scratch_types=[
            pltpu.VMEM((2, IDX_PAD // 2), jnp.int32),
            pltpu.VMEM((2, IDX_PAD // 2), jnp.int32),
            pltpu.VMEM((2, IDX_PAD // 2), jnp.float32),
            pltpu.VMEM_SHARED((HALF + 8,), jnp.float32),
            pltpu.SemaphoreType.DMA,
        ],
        compiler_params=pltpu.CompilerParams(use_tc_tiling_on_sc=False),
    )
    s = multihot(idx2, ones2, zeros_half)

    eT = embeddings.T
    wT = W.T

    logits = pl.pallas_call(
        _pool_project_body,
        grid=(2 * NBK,),
        in_specs=[
            pl.BlockSpec((BLK,), lambda i: (jnp.minimum(i, NBK - 1),)),
            pl.BlockSpec((EMBED, BLK),
                         lambda i: (0, jnp.minimum(i, NBK - 1))),
            pl.BlockSpec((EMBED, BLK),
                         lambda i: (0, jnp.maximum(i - NBK, 0))),
            pl.BlockSpec((BLK,), lambda i: (jnp.maximum(i - NBK, 0),)),
        ],
        out_specs=pl.BlockSpec((1, BLK),
                               lambda i: (0, jnp.maximum(i - NBK, 0))),
        out_shape=jax.ShapeDtypeStruct((1, VOCAB), jnp.float32),
        scratch_shapes=[pltpu.VMEM((1, EMBED), jnp.float32)],
    )(s, eT, wT, b)
    return logits


# speedup vs baseline: 1.2109x; 1.0036x over previous
import functools

import jax
import jax.numpy as jnp
from jax import lax
from jax.experimental import pallas as pl
from jax.experimental.pallas import tpu as pltpu
from jax.experimental.pallas import tpu_sc as plsc

VOCAB = 1_000_000
EMBED = 64
CTX = 200
HALF = VOCAB // 2
IDX_PAD = 224
BLK = 49_152
NBK = 21


def _sc_multihot_body(idx_hbm, ones_hbm, zeros_hbm, out_hbm,
                      idx_v, idxe_v, ones_v, spm, _sem):
    c = lax.axis_index("c")
    sid = lax.axis_index("s")

    @pl.when(sid == 0)
    def _():
        pltpu.sync_copy(zeros_hbm, spm.at[pl.ds(0, HALF)])
        pltpu.sync_copy(idx_hbm, idx_v)
        pltpu.sync_copy(ones_hbm, ones_v)
        base = c * HALF
        for j in range(2):
            for k in range(IDX_PAD // 32):
                sl = pl.ds(16 * k, 16)
                v = idx_v[j, sl]
                local = v - base
                valid = jnp.logical_and(local >= 0, local < HALF)
                idxe_v[j, sl] = jnp.where(valid, local, HALF)
        for j in range(2):
            pltpu.sync_copy(ones_v.at[j], spm.at[idxe_v.at[j]], add=True)
        pltpu.sync_copy(spm.at[pl.ds(0, HALF)], out_hbm.at[pl.ds(base, HALF)])


def _pool_project_body(s_ref, et_ref, wt_ref, b_ref, out_ref, acc_ref):
    i = pl.program_id(0)

    @pl.when(i < NBK)
    def _():
        sv = s_ref[...].reshape(1, BLK)
        limit = jnp.minimum(VOCAB - i * BLK, BLK)
        lane = lax.broadcasted_iota(jnp.int32, (1, BLK), 1)
        sv = jnp.where(lane < limit, sv, 0.0)

        def dot(ev):
            return lax.dot_general(sv, ev, (((1,), (1,)), ((), ())),
                                   preferred_element_type=jnp.float32)

        @pl.when(i == 0)
        def _():
            acc_ref[...] = dot(et_ref[...])

        @pl.when(jnp.logical_and(i > 0, i < NBK - 1))
        def _():
            acc_ref[...] += dot(et_ref[...])

        @pl.when(i == NBK - 1)
        def _():
            lane2 = lax.broadcasted_iota(jnp.int32, (EMBED, BLK), 1)
            ev = jnp.where(lane2 < limit, et_ref[...], 0.0)
            acc_ref[...] += dot(ev)

    @pl.when(i >= NBK)
    def _():
        avg = acc_ref[...] * (1.0 / CTX)
        out_ref[...] = lax.dot_general(
            avg, wt_ref[...], (((1,), (0,)), ((), ())),
            preferred_element_type=jnp.float32) + b_ref[...].reshape(1, BLK)


def kernel(context_idx, embeddings, W, b):
    ci = context_idx.astype(jnp.int32)
    idx2 = jnp.pad(ci, (0, IDX_PAD - CTX),
                   constant_values=VOCAB).reshape(2, IDX_PAD // 2)
    ones2 = jnp.ones((2, IDX_PAD // 2), jnp.float32)
    zeros_half = jnp.zeros((HALF,), jnp.float32)

    mesh = plsc.VectorSubcoreMesh(core_axis_name="c", subcore_axis_name="s")
    multihot = pl.kernel(
        _sc_multihot_body,
        mesh=mesh,
        out_type=jax.ShapeDtypeStruct((VOCAB,), jnp.float32),
        scratch_types=[
            pltpu.VMEM((2, IDX_PAD // 2), jnp.int32),
            pltpu.VMEM((2, IDX_PAD // 2), jnp.int32),
            pltpu.VMEM((2, IDX_PAD // 2), jnp.float32),
            pltpu.VMEM_SHARED((HALF + 8,), jnp.float32),
            pltpu.SemaphoreType.DMA,
        ],
        compiler_params=pltpu.CompilerParams(use_tc_tiling_on_sc=False),
    )
    s = multihot(idx2, ones2, zeros_half)

    eT = embeddings.T
    wT = W.T

    logits = pl.pallas_call(
        _pool_project_body,
        grid=(2 * NBK,),
        in_specs=[
            pl.BlockSpec((BLK,), lambda i: (jnp.minimum(i, NBK - 1),)),
            pl.BlockSpec((EMBED, BLK),
                         lambda i: (0, jnp.minimum(i, NBK - 1))),
            pl.BlockSpec((EMBED, BLK),
                         lambda i: (0, jnp.maximum(i - NBK, 0))),
            pl.BlockSpec((BLK,), lambda i: (jnp.maximum(i - NBK, 0),)),
        ],
        out_specs=pl.BlockSpec((1, BLK),
                               lambda i: (0, jnp.maximum(i - NBK, 0))),
        out_shape=jax.ShapeDtypeStruct((1, VOCAB), jnp.float32),
        scratch_shapes=[pltpu.VMEM((1, EMBED), jnp.float32)],
    )(s, eT, wT, b)
    return logits
